# baseline (speedup 1.0000x reference)
import functools

import jax
import jax.numpy as jnp
from jax import lax
from jax.experimental import pallas as pl
from jax.experimental.pallas import tpu as pltpu
from jax.experimental.pallas import tpu_sc as plsc

B = 16384
D = 64
NC = 2
NS = 16
NW = NC * NS
BPW = B // NW
IDX_ROWS = BPW // 128

_mesh = plsc.VectorSubcoreMesh(
    core_axis_name="c", subcore_axis_name="s", num_cores=NC, num_subcores=NS
)


@functools.partial(
    pl.kernel,
    out_type=jax.ShapeDtypeStruct((B // 128, 128), jnp.float32),
    mesh=_mesh,
    scratch_types=[
        pltpu.VMEM((IDX_ROWS, 128), jnp.int32),
        pltpu.VMEM((IDX_ROWS, 128), jnp.int32),
        pltpu.VMEM((IDX_ROWS, 128), jnp.float32),
        pltpu.VMEM((IDX_ROWS, 128), jnp.float32),
        pltpu.SemaphoreType.DMA,
    ],
    compiler_params=pltpu.CompilerParams(needs_layout_passes=False),
)
def _sc_bias_kernel(
    uid_hbm, iid_hbm, bu_hbm, bi_hbm,
    out_hbm,
    idx_u, idx_i, bu_v, bi_v, sem,
):
    wid = lax.axis_index("s") * NC + lax.axis_index("c")
    irow = wid * IDX_ROWS

    cp_u = pltpu.async_copy(uid_hbm.at[pl.ds(irow, IDX_ROWS)], idx_u, sem)
    cp_i = pltpu.async_copy(iid_hbm.at[pl.ds(irow, IDX_ROWS)], idx_i, sem)
    cp_u.wait()
    cp_i.wait()

    gathers = []
    for j in range(IDX_ROWS):
        gathers.append(pltpu.async_copy(bu_hbm.at[idx_u.at[j]], bu_v.at[j], sem))
        gathers.append(pltpu.async_copy(bi_hbm.at[idx_i.at[j]], bi_v.at[j], sem))
    for g in gathers:
        g.wait()

    for j in range(IDX_ROWS):
        for k in range(128 // 16):
            s = bu_v[j, pl.ds(16 * k, 16)] + bi_v[j, pl.ds(16 * k, 16)]
            bu_v[j, pl.ds(16 * k, 16)] = s

    pltpu.sync_copy(bu_v, out_hbm.at[pl.ds(irow, IDX_ROWS)])


_TC_BLK = 8192


def _tc_dense_body(uft_ref, ift_ref, w_ref, b_ref, o_ref):
    t = uft_ref[...] * ift_ref[...]
    s = jax.lax.dot_general(
        w_ref[...], t, (((1,), (0,)), ((), ())),
        preferred_element_type=jnp.float32,
    )
    o_ref[...] = (s + b_ref[0]).reshape(_TC_BLK // 128, 128)


_tc_dense = pl.pallas_call(
    _tc_dense_body,
    grid=(B // _TC_BLK,),
    in_specs=[
        pl.BlockSpec((D, _TC_BLK), lambda i: (0, i)),
        pl.BlockSpec((D, _TC_BLK), lambda i: (0, i)),
        pl.BlockSpec((1, D), lambda i: (0, 0)),
        pl.BlockSpec(memory_space=pltpu.SMEM),
    ],
    out_specs=pl.BlockSpec((_TC_BLK // 128, 128), lambda i: (i, 0)),
    out_shape=jax.ShapeDtypeStruct((B // 128, 128), jnp.float32),
)


def kernel(user_feature, user_id, item_feature, item_id, W, b, b_user, b_item):
    uid = user_id.astype(jnp.int32).reshape(B // 128, 128)
    iid = item_id.astype(jnp.int32).reshape(B // 128, 128)
    bias = _sc_bias_kernel(uid, iid, b_user, b_item)
    lin = _tc_dense(user_feature.T, item_feature.T, W, b)
    return (lin + bias).reshape(B, 1)


# speedup vs baseline: 1.0107x; 1.0107x over previous
import functools

import jax
import jax.numpy as jnp
from jax import lax
from jax.experimental import pallas as pl
from jax.experimental.pallas import tpu as pltpu
from jax.experimental.pallas import tpu_sc as plsc

B = 16384
D = 64
NC = 2
NS = 16
NW = NC * NS
BPW = B // NW
IDX_ROWS = BPW // 128

_mesh = plsc.VectorSubcoreMesh(
    core_axis_name="c", subcore_axis_name="s", num_cores=NC, num_subcores=NS
)


@functools.partial(
    pl.kernel,
    out_type=jax.ShapeDtypeStruct((B // 128, 128), jnp.float32),
    mesh=_mesh,
    scratch_types=[
        pltpu.VMEM((IDX_ROWS, 128), jnp.int32),
        pltpu.VMEM((IDX_ROWS, 128), jnp.int32),
        pltpu.VMEM((IDX_ROWS, 128), jnp.float32),
        pltpu.VMEM((IDX_ROWS, 128), jnp.float32),
        pltpu.SemaphoreType.DMA,
        pltpu.SemaphoreType.DMA((IDX_ROWS,)),
        pltpu.SemaphoreType.DMA,
    ],
    compiler_params=pltpu.CompilerParams(needs_layout_passes=False),
)
def _sc_bias_kernel(
    uid_hbm, iid_hbm, bu_hbm, bi_hbm,
    out_hbm,
    idx_u, idx_i, bu_v, bi_v, sem, sem_g, sem_o,
):
    wid = lax.axis_index("s") * NC + lax.axis_index("c")
    irow = wid * IDX_ROWS

    cp_u = pltpu.async_copy(uid_hbm.at[pl.ds(irow, IDX_ROWS)], idx_u, sem)
    cp_i = pltpu.async_copy(iid_hbm.at[pl.ds(irow, IDX_ROWS)], idx_i, sem)
    cp_u.wait()
    cp_i.wait()

    gathers = []
    for j in range(IDX_ROWS):
        gathers.append(
            pltpu.async_copy(bu_hbm.at[idx_u.at[j]], bu_v.at[j], sem_g.at[j])
        )
        gathers.append(
            pltpu.async_copy(bi_hbm.at[idx_i.at[j]], bi_v.at[j], sem_g.at[j])
        )

    outs = []
    for j in range(IDX_ROWS):
        gathers[2 * j].wait()
        gathers[2 * j + 1].wait()
        for k in range(128 // 16):
            s = bu_v[j, pl.ds(16 * k, 16)] + bi_v[j, pl.ds(16 * k, 16)]
            bu_v[j, pl.ds(16 * k, 16)] = s
        outs.append(
            pltpu.async_copy(bu_v.at[j], out_hbm.at[irow + j], sem_o)
        )
    for o in outs:
        o.wait()


_TC_BLK = 8192


def _tc_dense_body(uft_ref, ift_ref, w_ref, b_ref, o_ref):
    t = uft_ref[...] * ift_ref[...]
    s = jax.lax.dot_general(
        w_ref[...], t, (((1,), (0,)), ((), ())),
        preferred_element_type=jnp.float32,
    )
    o_ref[...] = (s + b_ref[0]).reshape(_TC_BLK // 128, 128)


_tc_dense = pl.pallas_call(
    _tc_dense_body,
    grid=(B // _TC_BLK,),
    in_specs=[
        pl.BlockSpec((D, _TC_BLK), lambda i: (0, i)),
        pl.BlockSpec((D, _TC_BLK), lambda i: (0, i)),
        pl.BlockSpec((1, D), lambda i: (0, 0)),
        pl.BlockSpec(memory_space=pltpu.SMEM),
    ],
    out_specs=pl.BlockSpec((_TC_BLK // 128, 128), lambda i: (i, 0)),
    out_shape=jax.ShapeDtypeStruct((B // 128, 128), jnp.float32),
)


def kernel(user_feature, user_id, item_feature, item_id, W, b, b_user, b_item):
    uid = user_id.astype(jnp.int32).reshape(B // 128, 128)
    iid = item_id.astype(jnp.int32).reshape(B // 128, 128)
    bias = _sc_bias_kernel(uid, iid, b_user, b_item)
    lin = _tc_dense(user_feature.T, item_feature.T, W, b)
    return (lin + bias).reshape(B, 1)
